# baseline (speedup 1.0000x reference)
import functools

import jax
import jax.numpy as jnp
from jax import lax
from jax.experimental import pallas as pl
from jax.experimental.pallas import tpu as pltpu
from jax.experimental.pallas import tpu_sc as plsc

_NC = 2
_NS = 16
_NW = _NC * _NS

_G = 128
_K = 4
_CHUNK = _G * _K


def _embedding_gather(idx2d, weight):
    n_rows, g = idx2d.shape
    v, d = weight.shape
    b = n_rows * g
    rows_per_w = n_rows // _NW
    n_iters = rows_per_w // _K

    mesh = plsc.VectorSubcoreMesh(core_axis_name="c", subcore_axis_name="s")

    @functools.partial(
        pl.kernel,
        out_type=jax.ShapeDtypeStruct((b, d), jnp.float32),
        mesh=mesh,
        compiler_params=pltpu.CompilerParams(use_tc_tiling_on_sc=False),
        scratch_types=[
            pltpu.VMEM((_K, _G), jnp.int32),
            pltpu.VMEM((_CHUNK, d), jnp.float32),
            pltpu.SemaphoreType.DMA,
        ],
    )
    def body(table_hbm, idx_hbm, out_hbm, idx_v, rows_v, sem):
        wid = lax.axis_index("s") * _NC + lax.axis_index("c")
        row_base = wid * rows_per_w

        def chunk(it, carry):
            row_off = row_base + it * _K
            pltpu.sync_copy(idx_hbm.at[pl.ds(row_off, _K)], idx_v)
            copies = []
            for j in range(_K):
                copies.append(
                    pltpu.async_copy(
                        table_hbm.at[idx_v.at[j]],
                        rows_v.at[pl.ds(j * _G, _G)],
                        sem,
                    )
                )
            for c in copies:
                c.wait()
            pltpu.sync_copy(rows_v, out_hbm.at[pl.ds(row_off * _G, _CHUNK)])
            return carry

        lax.fori_loop(0, n_iters, chunk, 0)

    return body(weight, idx2d)


def kernel(input_ids, weight):
    b0, s = input_ids.shape
    v, d = weight.shape
    idx2d = input_ids.astype(jnp.int32).reshape(-1, _G)
    out = _embedding_gather(idx2d, weight)
    return out.reshape(b0, s, d)


# speedup vs baseline: 1.0454x; 1.0454x over previous
import functools

import jax
import jax.numpy as jnp
from jax import lax
from jax.experimental import pallas as pl
from jax.experimental.pallas import tpu as pltpu
from jax.experimental.pallas import tpu_sc as plsc

_NC = 2
_NS = 16
_NW = _NC * _NS

_G = 128
_K = 4
_CHUNK = _G * _K
_NBUF = 2


def _embedding_gather(idx2d, weight):
    n_rows, g = idx2d.shape
    v, d = weight.shape
    b = n_rows * g
    rows_per_w = n_rows // _NW
    n_iters = rows_per_w // _K
    assert n_iters % _NBUF == 0 and n_iters >= 2 * _NBUF

    mesh = plsc.VectorSubcoreMesh(core_axis_name="c", subcore_axis_name="s")

    @functools.partial(
        pl.kernel,
        out_type=jax.ShapeDtypeStruct((b, d), jnp.float32),
        mesh=mesh,
        compiler_params=pltpu.CompilerParams(use_tc_tiling_on_sc=False),
        scratch_types=[
            pltpu.VMEM((rows_per_w, _G), jnp.int32),
            pltpu.VMEM((_NBUF, _CHUNK, d), jnp.float32),
            [pltpu.SemaphoreType.DMA] * _NBUF,
            [pltpu.SemaphoreType.DMA] * _NBUF,
        ],
    )
    def body(table_hbm, idx_hbm, out_hbm, idx_v, rows_v, gsems, wsems):
        wid = lax.axis_index("s") * _NC + lax.axis_index("c")
        row_base = wid * rows_per_w

        pltpu.sync_copy(idx_hbm.at[pl.ds(row_base, rows_per_w)], idx_v)

        def fire_gathers(it, bi):
            for j in range(_K):
                pltpu.async_copy(
                    table_hbm.at[idx_v.at[it * _K + j]],
                    rows_v.at[bi].at[pl.ds(j * _G, _G)],
                    gsems[bi],
                )

        def drain_gathers(bi):
            for j in range(_K):
                pltpu.make_async_copy(
                    table_hbm.at[idx_v.at[j]],
                    rows_v.at[bi].at[pl.ds(j * _G, _G)],
                    gsems[bi],
                ).wait()

        def fire_writeback(it, bi):
            return pltpu.async_copy(
                rows_v.at[bi],
                out_hbm.at[pl.ds((row_base + it * _K) * _G, _CHUNK)],
                wsems[bi],
            )

        def drain_writeback(it, bi):
            pltpu.make_async_copy(
                rows_v.at[bi],
                out_hbm.at[pl.ds((row_base + it * _K) * _G, _CHUNK)],
                wsems[bi],
            ).wait()

        for bi in range(_NBUF):
            fire_gathers(bi, bi)

        def outer(i2, carry):
            for bi in range(_NBUF):
                it = i2 * _NBUF + bi
                drain_gathers(bi)
                fire_writeback(it, bi)
                drain_writeback(it, bi)
                fire_gathers(it + _NBUF, bi)
            return carry

        lax.fori_loop(0, (n_iters - _NBUF) // _NBUF, outer, 0)

        for bi in range(_NBUF):
            it = n_iters - _NBUF + bi
            drain_gathers(bi)
            fire_writeback(it, bi)
        for bi in range(_NBUF):
            it = n_iters - _NBUF + bi
            drain_writeback(it, bi)

    return body(weight, idx2d)


def kernel(input_ids, weight):
    b0, s = input_ids.shape
    v, d = weight.shape
    idx2d = input_ids.astype(jnp.int32).reshape(-1, _G)
    out = _embedding_gather(idx2d, weight)
    return out.reshape(b0, s, d)
